# Initial kernel scaffold; baseline (speedup 1.0000x reference)
import jax
import jax.numpy as jnp
from jax.experimental import pallas as pl


def kernel(merchant_id, mcc, currency, amount_bin, day_of_week, emb_merchant_id, emb_mcc, emb_currency, emb_amount_bin, emb_day_of_week, W, b):
    raise NotImplementedError("write your pallas kernel here")


# speedup vs baseline: 3.6176x; 3.6176x over previous
import functools

import jax
import jax.numpy as jnp
from jax import lax
from jax.experimental import pallas as pl
from jax.experimental.pallas import tpu as pltpu
from jax.experimental.pallas import tpu_sc as plsc

B, L = 4096, 50
N = B * L
D = 128
NC, NS = 2, 16
NW = NC * NS
TPW = N // NW
C = 128
NCHUNK = TPW // C

MERCHANT_BM = 2048


def _merchant_proj_kernel(a_ref, w_ref, b_ref, o_ref):
    o_ref[...] = (
        jnp.dot(a_ref[...], w_ref[...], preferred_element_type=jnp.float32)
        + b_ref[...]
    )


def _small_proj_kernel(t1, w1, t2, w2, t3, w3, t4, w4, o1, o2, o3, o4):
    o1[...] = jnp.dot(t1[...], w1[...], preferred_element_type=jnp.float32)
    o2[...] = jnp.dot(t2[...], w2[...], preferred_element_type=jnp.float32)
    o3[...] = jnp.dot(t3[...], w3[...], preferred_element_type=jnp.float32)
    o4[...] = jnp.dot(t4[...], w4[...], preferred_element_type=jnp.float32)


_sc_mesh = plsc.VectorSubcoreMesh(core_axis_name="c", subcore_axis_name="s")


@functools.partial(
    pl.kernel,
    out_type=jax.ShapeDtypeStruct((N, D), jnp.float32),
    mesh=_sc_mesh,
    scratch_types=[
        pltpu.VMEM((C,), jnp.int32),
        pltpu.VMEM((C,), jnp.int32),
        pltpu.VMEM((C,), jnp.int32),
        pltpu.VMEM((C,), jnp.int32),
        pltpu.VMEM((C,), jnp.int32),
        pltpu.VMEM((C, D), jnp.float32),
        pltpu.VMEM((C, D), jnp.float32),
        pltpu.VMEM((C, D), jnp.float32),
        pltpu.VMEM((C, D), jnp.float32),
        pltpu.VMEM((C, D), jnp.float32),
        pltpu.SemaphoreType.DMA,
    ],
)
def _sc_gather_sum(pm, pmcc, pcur, pamt, pdow,
                   im, imcc, icur, iamt, idow,
                   out_hbm,
                   iv0, iv1, iv2, iv3, iv4,
                   r0, r1, r2, r3, r4, sem):
    wid = lax.axis_index("s") * NC + lax.axis_index("c")
    base0 = wid * TPW

    def chunk_body(ci, carry):
        base = base0 + ci * C
        sl = pl.ds(base, C)
        pltpu.sync_copy(im.at[sl], iv0)
        pltpu.sync_copy(imcc.at[sl], iv1)
        pltpu.sync_copy(icur.at[sl], iv2)
        pltpu.sync_copy(iamt.at[sl], iv3)
        pltpu.sync_copy(idow.at[sl], iv4)
        cp0 = pltpu.async_copy(pm.at[iv0], r0, sem)
        cp1 = pltpu.async_copy(pmcc.at[iv1], r1, sem)
        cp2 = pltpu.async_copy(pcur.at[iv2], r2, sem)
        cp3 = pltpu.async_copy(pamt.at[iv3], r3, sem)
        cp4 = pltpu.async_copy(pdow.at[iv4], r4, sem)
        cp0.wait()
        cp1.wait()
        cp2.wait()
        cp3.wait()
        cp4.wait()

        def row_body(c2, carry2):
            for j in range(D // 16):
                s16 = pl.ds(j * 16, 16)
                v = (r0[c2, s16] + r1[c2, s16] + r2[c2, s16]
                     + r3[c2, s16] + r4[c2, s16])
                r0[c2, s16] = v
            return carry2

        lax.fori_loop(0, C, row_body, 0, unroll=2)
        pltpu.sync_copy(r0, out_hbm.at[sl])
        return carry

    lax.fori_loop(0, NCHUNK, chunk_body, 0)


def kernel(merchant_id, mcc, currency, amount_bin, day_of_week,
           emb_merchant_id, emb_mcc, emb_currency, emb_amount_bin,
           emb_day_of_week, W, b):
    w_m = W[0:64]
    w_mcc = W[64:96]
    w_cur = W[96:112]
    w_amt = W[112:144]
    w_dow = W[144:160]

    n_m = emb_merchant_id.shape[0]
    grid_m = (n_m + MERCHANT_BM - 1) // MERCHANT_BM
    pm = pl.pallas_call(
        _merchant_proj_kernel,
        grid=(grid_m,),
        in_specs=[
            pl.BlockSpec((MERCHANT_BM, 64), lambda i: (i, 0)),
            pl.BlockSpec((64, D), lambda i: (0, 0)),
            pl.BlockSpec((1, D), lambda i: (0, 0)),
        ],
        out_specs=pl.BlockSpec((MERCHANT_BM, D), lambda i: (i, 0)),
        out_shape=jax.ShapeDtypeStruct((n_m, D), jnp.float32),
    )(emb_merchant_id, w_m, b.reshape(1, D))

    small_outs = pl.pallas_call(
        _small_proj_kernel,
        out_shape=[
            jax.ShapeDtypeStruct((emb_mcc.shape[0], D), jnp.float32),
            jax.ShapeDtypeStruct((emb_currency.shape[0], D), jnp.float32),
            jax.ShapeDtypeStruct((emb_amount_bin.shape[0], D), jnp.float32),
            jax.ShapeDtypeStruct((emb_day_of_week.shape[0], D), jnp.float32),
        ],
    )(emb_mcc, w_mcc, emb_currency, w_cur,
      emb_amount_bin, w_amt, emb_day_of_week, w_dow)
    pmcc, pcur, pamt, pdow = small_outs

    out_flat = _sc_gather_sum(
        pm, pmcc, pcur, pamt, pdow,
        merchant_id.reshape(N), mcc.reshape(N), currency.reshape(N),
        amount_bin.reshape(N), day_of_week.reshape(N))
    return out_flat.reshape(B, L, D)
